# Initial kernel scaffold; baseline (speedup 1.0000x reference)
import functools

import jax
import jax.numpy as jnp
from jax import lax
from jax.experimental import pallas as pl
from jax.experimental.pallas import tpu as pltpu, tpu_sc as plsc

BATCH = 4096
HIST = 50
EMBED = 128
TOTAL = BATCH * HIST

NUM_CORES = 2
NUM_SUBCORES = 16
NUM_WORKERS = NUM_CORES * NUM_SUBCORES

ROWS_PER_WORKER = TOTAL // NUM_WORKERS
CHUNK = 128
NCHUNKS = ROWS_PER_WORKER // CHUNK

_mesh = plsc.VectorSubcoreMesh(core_axis_name="c", subcore_axis_name="s")


@functools.partial(
    pl.kernel,
    mesh=_mesh,
    out_type=jax.ShapeDtypeStruct((TOTAL, EMBED), jnp.float32),
    scratch_types=[
        pltpu.VMEM((NCHUNKS, CHUNK), jnp.int32),
        pltpu.VMEM((CHUNK, EMBED), jnp.float32),
        pltpu.SemaphoreType.DMA,
    ],
)
def _gather_sc(idx_hbm, table_hbm, out_hbm, idx_v, rows_v, sem):
    wid = lax.axis_index("s") * NUM_CORES + lax.axis_index("c")
    base = wid * ROWS_PER_WORKER
    pltpu.sync_copy(idx_hbm.at[pl.ds(wid * NCHUNKS, NCHUNKS)], idx_v)

    def body(j, carry):
        pltpu.async_copy(table_hbm.at[idx_v.at[j]], rows_v, sem).wait()
        pltpu.sync_copy(rows_v, out_hbm.at[pl.ds(base + j * CHUNK, CHUNK)])
        return carry

    lax.fori_loop(0, NCHUNKS, body, 0)


def kernel(indices, table):
    idx = indices.reshape(NUM_WORKERS * NCHUNKS, CHUNK).astype(jnp.int32)
    out = _gather_sc(idx, table)
    return out.reshape(BATCH, HIST, EMBED)


# speedup vs baseline: 2.9647x; 2.9647x over previous
import functools

import jax
import jax.numpy as jnp
from jax import lax
from jax.experimental import pallas as pl
from jax.experimental.pallas import tpu as pltpu, tpu_sc as plsc

BATCH = 4096
HIST = 50
EMBED = 128
TOTAL = BATCH * HIST

NUM_CORES = 2
NUM_SUBCORES = 16
NUM_WORKERS = NUM_CORES * NUM_SUBCORES

ROWS_PER_WORKER = TOTAL // NUM_WORKERS
CHUNK = 128
NCHUNKS = ROWS_PER_WORKER // CHUNK

_mesh = plsc.VectorSubcoreMesh(core_axis_name="c", subcore_axis_name="s")


@functools.partial(
    pl.kernel,
    mesh=_mesh,
    out_type=jax.ShapeDtypeStruct((TOTAL, EMBED), jnp.float32),
    scratch_types=[
        pltpu.VMEM((NCHUNKS, CHUNK), jnp.int32),
        pltpu.VMEM((CHUNK, EMBED), jnp.float32),
        pltpu.SemaphoreType.DMA,
    ],
)
def _gather_sc(idx_hbm, table_hbm, out_hbm, idx_v, rows_v, sem):
    wid = lax.axis_index("s") * NUM_CORES + lax.axis_index("c")
    base = wid * ROWS_PER_WORKER
    pltpu.sync_copy(idx_hbm.at[wid], idx_v)

    def body(j, carry):
        pltpu.async_copy(table_hbm.at[idx_v.at[j]], rows_v, sem).wait()
        pltpu.sync_copy(rows_v, out_hbm.at[pl.ds(base + j * CHUNK, CHUNK)])
        return carry

    lax.fori_loop(0, NCHUNKS, body, 0)


def kernel(indices, table):
    idx = indices.reshape(NUM_WORKERS, NCHUNKS, CHUNK).astype(jnp.int32)
    out = _gather_sc(idx, table)
    return out.reshape(BATCH, HIST, EMBED)


# speedup vs baseline: 3.3470x; 1.1290x over previous
import functools

import jax
import jax.numpy as jnp
from jax import lax
from jax.experimental import pallas as pl
from jax.experimental.pallas import tpu as pltpu, tpu_sc as plsc

BATCH = 4096
HIST = 50
EMBED = 128
TOTAL = BATCH * HIST

NUM_CORES = 2
NUM_SUBCORES = 16
NUM_WORKERS = NUM_CORES * NUM_SUBCORES

ROWS_PER_WORKER = TOTAL // NUM_WORKERS
CHUNK = 128
NCHUNKS = ROWS_PER_WORKER // CHUNK
NBUF = 5
NLAPS = NCHUNKS // NBUF

_mesh = plsc.VectorSubcoreMesh(core_axis_name="c", subcore_axis_name="s")


@functools.partial(
    pl.kernel,
    mesh=_mesh,
    out_type=jax.ShapeDtypeStruct((TOTAL, EMBED), jnp.float32),
    scratch_types=(
        [pltpu.VMEM((NCHUNKS, CHUNK), jnp.int32)]
        + [pltpu.VMEM((CHUNK, EMBED), jnp.float32) for _ in range(NBUF)]
        + [pltpu.SemaphoreType.DMA for _ in range(2 * NBUF)]
    ),
)
def _gather_sc(idx_hbm, table_hbm, out_hbm, idx_v, *rest):
    rows = rest[:NBUF]
    gsem = rest[NBUF : 2 * NBUF]
    ssem = rest[2 * NBUF : 3 * NBUF]

    wid = lax.axis_index("s") * NUM_CORES + lax.axis_index("c")
    base = wid * ROWS_PER_WORKER
    pltpu.sync_copy(idx_hbm.at[wid], idx_v)

    def gather_start(b, j):
        pltpu.async_copy(table_hbm.at[idx_v.at[j]], rows[b], gsem[b])

    def gather_wait(b):
        pltpu.make_async_copy(table_hbm.at[idx_v.at[0]], rows[b], gsem[b]).wait()

    def store_start(b, j):
        pltpu.async_copy(rows[b], out_hbm.at[pl.ds(base + j * CHUNK, CHUNK)], ssem[b])

    def store_wait(b):
        pltpu.make_async_copy(rows[b], out_hbm.at[pl.ds(base, CHUNK)], ssem[b]).wait()

    for b in range(NBUF):
        gather_start(b, b)

    def lap(g, carry):
        for b in range(NBUF):
            j = g * NBUF + b
            gather_wait(b)
            store_start(b, j)
            store_wait(b)
            gather_start(b, j + NBUF)
        return carry

    lax.fori_loop(0, NLAPS - 1, lap, 0)

    for b in range(NBUF):
        gather_wait(b)
        store_start(b, NCHUNKS - NBUF + b)
    for b in range(NBUF):
        store_wait(b)


def kernel(indices, table):
    idx = indices.reshape(NUM_WORKERS, NCHUNKS, CHUNK).astype(jnp.int32)
    out = _gather_sc(idx, table)
    return out.reshape(BATCH, HIST, EMBED)


# speedup vs baseline: 5.9300x; 1.7717x over previous
import functools

import jax
import jax.numpy as jnp
from jax import lax
from jax.experimental import pallas as pl
from jax.experimental.pallas import tpu as pltpu, tpu_sc as plsc

BATCH = 4096
HIST = 50
EMBED = 128

NUM_CORES = 2
NUM_SUBCORES = 16
NUM_WORKERS = NUM_CORES * NUM_SUBCORES

BPW = BATCH // NUM_WORKERS
CB = 2
NCHUNKS = BPW // CB
NBUF = 4
NLAPS = NCHUNKS // NBUF

_mesh = plsc.VectorSubcoreMesh(core_axis_name="c", subcore_axis_name="s")


@functools.partial(
    pl.kernel,
    mesh=_mesh,
    out_type=jax.ShapeDtypeStruct((BATCH, HIST, EMBED), jnp.float32),
    scratch_types=(
        [pltpu.VMEM((BPW, HIST), jnp.int32)]
        + [pltpu.VMEM((CB, HIST, EMBED), jnp.float32) for _ in range(NBUF)]
        + [pltpu.SemaphoreType.DMA for _ in range(2 * NBUF)]
    ),
)
def _gather_sc(idx_hbm, table_hbm, out_hbm, idx_v, *rest):
    rows = rest[:NBUF]
    gsem = rest[NBUF : 2 * NBUF]
    ssem = rest[2 * NBUF : 3 * NBUF]

    wid = lax.axis_index("s") * NUM_CORES + lax.axis_index("c")
    bbase = wid * BPW
    pltpu.sync_copy(idx_hbm.at[pl.ds(bbase, BPW)], idx_v)

    def gather_start(b, t):
        for i in range(CB):
            pltpu.async_copy(
                table_hbm.at[idx_v.at[t * CB + i]], rows[b].at[i], gsem[b]
            )

    def gather_wait(b):
        for i in range(CB):
            pltpu.make_async_copy(
                table_hbm.at[idx_v.at[0]], rows[b].at[i], gsem[b]
            ).wait()

    def store_start(b, t):
        pltpu.async_copy(rows[b], out_hbm.at[pl.ds(bbase + t * CB, CB)], ssem[b])

    def store_wait(b):
        pltpu.make_async_copy(rows[b], out_hbm.at[pl.ds(bbase, CB)], ssem[b]).wait()

    for b in range(NBUF):
        gather_start(b, b)

    def lap(g, carry):
        for b in range(NBUF):
            t = g * NBUF + b
            gather_wait(b)
            store_start(b, t)
            store_wait(b)
            gather_start(b, t + NBUF)
        return carry

    lax.fori_loop(0, NLAPS - 1, lap, 0)

    for b in range(NBUF):
        gather_wait(b)
        store_start(b, NCHUNKS - NBUF + b)
    for b in range(NBUF):
        store_wait(b)


def kernel(indices, table):
    return _gather_sc(indices.astype(jnp.int32), table)


# speedup vs baseline: 10.7562x; 1.8138x over previous
import functools

import jax
import jax.numpy as jnp
from jax import lax
from jax.experimental import pallas as pl
from jax.experimental.pallas import tpu as pltpu, tpu_sc as plsc

BATCH = 4096
HIST = 50
EMBED = 128

NUM_CORES = 2
NUM_SUBCORES = 16
NUM_WORKERS = NUM_CORES * NUM_SUBCORES

BPW = BATCH // NUM_WORKERS
NBUF = 5
NLAPS = HIST // NBUF

_mesh = plsc.VectorSubcoreMesh(core_axis_name="c", subcore_axis_name="s")


@functools.partial(
    pl.kernel,
    mesh=_mesh,
    out_type=jax.ShapeDtypeStruct((HIST, BATCH, EMBED), jnp.float32),
    scratch_types=(
        [pltpu.VMEM((HIST, BPW), jnp.int32)]
        + [pltpu.VMEM((BPW, EMBED), jnp.float32) for _ in range(NBUF)]
        + [pltpu.SemaphoreType.DMA for _ in range(2 * NBUF)]
    ),
)
def _gather_sc(idx_hbm, table_hbm, out_hbm, idx_v, *rest):
    rows = rest[:NBUF]
    gsem = rest[NBUF : 2 * NBUF]
    ssem = rest[2 * NBUF : 3 * NBUF]

    wid = lax.axis_index("s") * NUM_CORES + lax.axis_index("c")
    bbase = wid * BPW
    pltpu.sync_copy(idx_hbm.at[:, pl.ds(bbase, BPW)], idx_v)

    def gather_start(b, h):
        pltpu.async_copy(table_hbm.at[idx_v.at[h]], rows[b], gsem[b])

    def gather_wait(b):
        pltpu.make_async_copy(table_hbm.at[idx_v.at[0]], rows[b], gsem[b]).wait()

    def store_start(b, h):
        pltpu.async_copy(rows[b], out_hbm.at[h, pl.ds(bbase, BPW)], ssem[b])

    def store_wait(b):
        pltpu.make_async_copy(rows[b], out_hbm.at[0, pl.ds(bbase, BPW)], ssem[b]).wait()

    for b in range(NBUF):
        gather_start(b, b)

    def lap(g, carry):
        for b in range(NBUF):
            h = g * NBUF + b
            gather_wait(b)
            store_start(b, h)
            store_wait(b)
            gather_start(b, h + NBUF)
        return carry

    lax.fori_loop(0, NLAPS - 1, lap, 0)

    for b in range(NBUF):
        gather_wait(b)
        store_start(b, HIST - NBUF + b)
    for b in range(NBUF):
        store_wait(b)


def kernel(indices, table):
    idx_t = indices.astype(jnp.int32).T
    out = _gather_sc(idx_t, table)
    return out.transpose(1, 0, 2)
